# baseline (speedup 1.0000x reference)
import jax
import jax.numpy as jnp
import numpy as np
from jax import lax
from jax.experimental import pallas as pl
from jax.experimental.pallas import tpu as pltpu

jax.config.update("jax_default_matmul_precision", "float32")

RC = 5.0
NB = 16
CC = 64
NQ = 9
CMT = 576
BI = 32
OFFS = (0, 64, 256, 576)
MS = (1, 3, 5)
WID = RC / NB


def _silu(x):
    return x * jax.nn.sigmoid(x)


def _dsilu(x):
    s = jax.nn.sigmoid(x)
    return s * (1.0 + x * (1.0 - s))


def _pair_quantities(posn_ref, post, nmask, i0, bi, n):
    xi = posn_ref[0, pl.ds(i0, bi), 0:1]
    yi = posn_ref[0, pl.ds(i0, bi), 1:2]
    zi = posn_ref[0, pl.ds(i0, bi), 2:3]
    dx = post[0:1, :] - xi
    dy = post[1:2, :] - yi
    dz = post[2:3, :] - zi
    d2 = dx * dx + dy * dy + dz * dz + 1e-12
    dd = jnp.sqrt(d2)
    inv = 1.0 / dd
    row = lax.broadcasted_iota(jnp.int32, (bi, n), 0) + i0
    col = lax.broadcasted_iota(jnp.int32, (bi, n), 1)
    offdiag = jnp.where(row == col, 0.0, 1.0).astype(jnp.float32)
    incut = jnp.where(dd < RC, 1.0, 0.0).astype(jnp.float32)
    maskf = nmask * offdiag * incut
    env = 0.5 * (jnp.cos(jnp.pi * jnp.minimum(dd / RC, 1.0)) + 1.0) * maskf
    cent = (lax.broadcasted_iota(jnp.int32, (NB, 1, 1), 0).astype(jnp.float32)
            * np.float32(RC / (NB - 1)))
    dev = dd[None, :, :] - cent
    rb = jnp.exp(-(dev * dev) / (2.0 * WID * WID))
    P = rb * env[None, :, :]
    xh = dx * inv
    yh = dy * inv
    zh = dz * inv
    Ys = jnp.concatenate(
        [
            jnp.ones((1, bi, n), jnp.float32),
            xh[None], yh[None], zh[None],
            (xh * yh)[None], (yh * zh)[None],
            (3.0 * zh * zh - 1.0)[None],
            (xh * zh)[None], (xh * xh - yh * yh)[None],
        ],
        axis=0,
    )
    return dict(dd=dd, inv=inv, maskf=maskf, env=env, rb=rb, P=P,
                xh=xh, yh=yh, zh=zh, Ys=Ys, dev=dev)


def _fwd_body(z_ref, embp_ref, posn_ref, post_ref, nmask_ref, wexp_ref,
              vexp_ref, lw0_ref, lw1_ref, lw2_ref, lb_ref, ow_ref,
              fin_ref, u_ref, a_ref, e_ref, fbuf):
    lg = pl.program_id(1)
    ib = pl.program_id(2)
    i0 = ib * BI
    n = fbuf.shape[1]

    @pl.when(jnp.logical_and(lg == 0, ib == 0))
    def _():
        zc = z_ref[0]
        io = lax.broadcasted_iota(jnp.int32, (n, 128), 1)
        oh = jnp.where(io == zc, 1.0, 0.0).astype(jnp.float32)
        f0 = jnp.dot(oh, embp_ref[...], preferred_element_type=jnp.float32, precision=lax.Precision.HIGHEST)
        fbuf[0, :, 0:CC] = f0
        fbuf[0, :, CC:CMT] = jnp.zeros((n, CMT - CC), jnp.float32)

    cur = lax.rem(lg, 2)
    nxt = lax.rem(lg + 1, 2)
    post = post_ref[0]
    nm = nmask_ref[0]
    F = fbuf[cur]
    pq = _pair_quantities(posn_ref, post, nm, i0, BI, n)
    P, Ys = pq["P"], pq["Ys"]
    PY = P[:, None, :, :] * Ys[None, :, :, :]
    G = jnp.dot(P.reshape(NB * BI, n), F,
                preferred_element_type=jnp.float32, precision=lax.Precision.HIGHEST)
    H = jnp.dot(PY.reshape(NB * NQ * BI, n), F[:, 0:CC],
                preferred_element_type=jnp.float32, precision=lax.Precision.HIGHEST)
    t1 = jnp.sum(G.reshape(NB, BI, CMT) * wexp_ref[0][:, None, :], axis=0)
    H4 = H.reshape(NB, NQ, BI, CC)
    t2q = jnp.sum(H4 * vexp_ref[0][:, :, None, :], axis=0)
    t2 = jnp.concatenate([t2q[q] for q in range(NQ)], axis=1)
    conv = t1 + t2
    Fi = fbuf[cur, pl.ds(i0, BI), :]
    lws = (lw0_ref, lw1_ref, lw2_ref)
    nf0 = None
    for l in range(3):
        sl = slice(OFFS[l], OFFS[l + 1])
        u = jnp.dot(conv[:, sl], lws[l][0],
                    preferred_element_type=jnp.float32, precision=lax.Precision.HIGHEST) + lb_ref[0, 0:1, sl]
        y = _silu(u)
        a = Fi[:, sl] + y
        nf = _silu(a)
        u_ref[0, 0, :, sl] = u
        a_ref[0, 0, :, sl] = a
        fbuf[nxt, pl.ds(i0, BI), sl] = nf
        if l == 0:
            nf0 = nf
    fin_ref[0, 0, :, :] = Fi
    eval_ = jnp.sum(nf0 * ow_ref[...])

    @pl.when(jnp.logical_and(lg == 2, ib == 0))
    def _():
        e_ref[...] = jnp.full((1, 1, 1), 0.0, jnp.float32) + eval_

    @pl.when(jnp.logical_and(lg == 2, ib > 0))
    def _():
        e_ref[...] = e_ref[...] + eval_


def _bwd_body(posn_ref, post_ref, nmask_ref, ft_ref, u_ref, a_ref,
              gn3_ref, wexp_ref, vexp_ref, lwt0_ref, lwt1_ref, lwt2_ref,
              gpt_ref, gbuf):
    lg = pl.program_id(1)
    ib = pl.program_id(2)
    i0 = ib * BI
    n = gbuf.shape[1]

    @pl.when(jnp.logical_and(lg == 0, ib == 0))
    def _():
        gbuf[0, :, :] = gn3_ref[0]
        gpt_ref[0, :, :] = jnp.zeros((3, n), jnp.float32)

    @pl.when(ib == 0)
    def _():
        gbuf[lax.rem(lg + 1, 2), :, :] = jnp.zeros((n, CMT), jnp.float32)

    cur = lax.rem(lg, 2)
    nxt = lax.rem(lg + 1, 2)
    post = post_ref[0]
    nm = nmask_ref[0]
    FT = ft_ref[0, 0]
    pq = _pair_quantities(posn_ref, post, nm, i0, BI, n)
    P, Ys = pq["P"], pq["Ys"]

    a = a_ref[0, 0]
    u = u_ref[0, 0]
    gn = gbuf[cur, pl.ds(i0, BI), :]
    ga = gn * _dsilu(a)
    gu = ga * _dsilu(u)
    lwts = (lwt0_ref, lwt1_ref, lwt2_ref)
    gconv = jnp.concatenate(
        [jnp.dot(gu[:, OFFS[l]:OFFS[l + 1]], lwts[l][0],
                 preferred_element_type=jnp.float32, precision=lax.Precision.HIGHEST) for l in range(3)],
        axis=1,
    )

    U1 = wexp_ref[0][:, None, :] * gconv[None]
    gP1 = jnp.dot(U1.reshape(NB * BI, CMT), FT,
                  preferred_element_type=jnp.float32, precision=lax.Precision.HIGHEST)
    gconvq = jnp.concatenate(
        [gconv[None, :, q * CC:(q + 1) * CC] for q in range(NQ)], axis=0)
    U2 = vexp_ref[0][:, :, None, :] * gconvq[None]
    Z = jnp.dot(U2.reshape(NB * NQ * BI, CC), FT[0:CC, :],
                preferred_element_type=jnp.float32, precision=lax.Precision.HIGHEST)
    Z4 = Z.reshape(NB, NQ, BI, n)
    gP = gP1.reshape(NB, BI, n) + jnp.sum(Z4 * Ys[None], axis=1)
    gY = jnp.sum(Z4 * P[:, None, :, :], axis=0)

    @pl.when(lg < 2)
    def _():
        gf1 = lax.dot_general(P.reshape(NB * BI, n), U1.reshape(NB * BI, CMT),
                              (((0,), (0,)), ((), ())),
                              preferred_element_type=jnp.float32, precision=lax.Precision.HIGHEST)
        PY = P[:, None, :, :] * Ys[None]
        gs = lax.dot_general(PY.reshape(NB * NQ * BI, n),
                             U2.reshape(NB * NQ * BI, CC),
                             (((0,), (0,)), ((), ())),
                             preferred_element_type=jnp.float32, precision=lax.Precision.HIGHEST)
        gbuf[nxt, :, :] += gf1
        gbuf[nxt, :, 0:CC] += gs
        gbuf[nxt, pl.ds(i0, BI), :] += ga

    dd, inv, env, rb, maskf = pq["dd"], pq["inv"], pq["env"], pq["rb"], pq["maskf"]
    xh, yh, zh, dev = pq["xh"], pq["yh"], pq["zh"], pq["dev"]
    rbp = rb * (-dev / (WID * WID))
    envp = (-0.5 * jnp.pi / RC) * jnp.sin(
        jnp.pi * jnp.minimum(dd / RC, 1.0)) * maskf
    gd = jnp.sum(gP * (rbp * env[None] + rb * envp[None]), axis=0)
    gxh = gY[1] + gY[4] * yh + gY[7] * zh + 2.0 * gY[8] * xh
    gyh = gY[2] + gY[4] * xh + gY[5] * zh - 2.0 * gY[8] * yh
    gzh = gY[3] + gY[5] * yh + 6.0 * gY[6] * zh + gY[7] * xh
    dotg = gxh * xh + gyh * yh + gzh * zh
    grx = gd * xh + (gxh - dotg * xh) * inv
    gry = gd * yh + (gyh - dotg * yh) * inv
    grz = gd * zh + (gzh - dotg * zh) * inv

    ones_bi = jnp.ones((1, BI), jnp.float32)
    ones_n = jnp.ones((1, n), jnp.float32)
    cn = (((1,), (1,)), ((), ()))
    colx = jnp.dot(ones_bi, grx, preferred_element_type=jnp.float32, precision=lax.Precision.HIGHEST)
    coly = jnp.dot(ones_bi, gry, preferred_element_type=jnp.float32, precision=lax.Precision.HIGHEST)
    colz = jnp.dot(ones_bi, grz, preferred_element_type=jnp.float32, precision=lax.Precision.HIGHEST)
    rowx = lax.dot_general(ones_n, grx, cn,
                           preferred_element_type=jnp.float32, precision=lax.Precision.HIGHEST)
    rowy = lax.dot_general(ones_n, gry, cn, preferred_element_type=jnp.float32, precision=lax.Precision.HIGHEST)
    rowz = lax.dot_general(ones_n, grz, cn, preferred_element_type=jnp.float32, precision=lax.Precision.HIGHEST)
    ri = lax.broadcasted_iota(jnp.int32, (BI, n), 0) + i0
    ci = lax.broadcasted_iota(jnp.int32, (BI, n), 1)
    oh = jnp.where(ri == ci, 1.0, 0.0).astype(jnp.float32)
    sx = jnp.dot(rowx, oh, preferred_element_type=jnp.float32, precision=lax.Precision.HIGHEST)
    sy = jnp.dot(rowy, oh, preferred_element_type=jnp.float32, precision=lax.Precision.HIGHEST)
    sz = jnp.dot(rowz, oh, preferred_element_type=jnp.float32, precision=lax.Precision.HIGHEST)

    gpt_ref[0, 0:1, :] += colx - sx
    gpt_ref[0, 1:2, :] += coly - sy
    gpt_ref[0, 2:3, :] += colz - sz


def _fwd_all(z3, embp, posn, post, nmaskf, wexp, vexp, lw0, lw1, lw2, lb, ow):
    B, N = posn.shape[0], posn.shape[1]
    NI = N // BI
    return pl.pallas_call(
        _fwd_body,
        grid=(B, 3, NI),
        in_specs=[
            pl.BlockSpec((1, N, 1), lambda b, l, i: (b, 0, 0)),
            pl.BlockSpec((128, CC), lambda b, l, i: (0, 0)),
            pl.BlockSpec((1, N, 3), lambda b, l, i: (b, 0, 0)),
            pl.BlockSpec((1, 3, N), lambda b, l, i: (b, 0, 0)),
            pl.BlockSpec((1, BI, N), lambda b, l, i: (b, i, 0)),
            pl.BlockSpec((1, NB, CMT), lambda b, l, i: (l, 0, 0)),
            pl.BlockSpec((1, NB, NQ, CC), lambda b, l, i: (l, 0, 0, 0)),
            pl.BlockSpec((1, 64, 64), lambda b, l, i: (l, 0, 0)),
            pl.BlockSpec((1, 192, 192), lambda b, l, i: (l, 0, 0)),
            pl.BlockSpec((1, 320, 320), lambda b, l, i: (l, 0, 0)),
            pl.BlockSpec((1, 1, CMT), lambda b, l, i: (l, 0, 0)),
            pl.BlockSpec((1, CC), lambda b, l, i: (0, 0)),
        ],
        out_specs=[
            pl.BlockSpec((1, 1, BI, CMT), lambda b, l, i: (b, l, i, 0)),
            pl.BlockSpec((1, 1, BI, CMT), lambda b, l, i: (b, l, i, 0)),
            pl.BlockSpec((1, 1, BI, CMT), lambda b, l, i: (b, l, i, 0)),
            pl.BlockSpec((1, 1, 1), lambda b, l, i: (b, 0, 0)),
        ],
        out_shape=[
            jax.ShapeDtypeStruct((B, 3, N, CMT), jnp.float32),
            jax.ShapeDtypeStruct((B, 3, N, CMT), jnp.float32),
            jax.ShapeDtypeStruct((B, 3, N, CMT), jnp.float32),
            jax.ShapeDtypeStruct((B, 1, 1), jnp.float32),
        ],
        scratch_shapes=[pltpu.VMEM((2, N, CMT), jnp.float32)],
    )(z3, embp, posn, post, nmaskf, wexp, vexp, lw0, lw1, lw2, lb, ow)


def _bwd_all(posn, post, nmaskf, fts, U, A, gn3, wexp, vexp, lwt0, lwt1, lwt2):
    B, N = posn.shape[0], posn.shape[1]
    NI = N // BI
    return pl.pallas_call(
        _bwd_body,
        grid=(B, 3, NI),
        in_specs=[
            pl.BlockSpec((1, N, 3), lambda b, l, i: (b, 0, 0)),
            pl.BlockSpec((1, 3, N), lambda b, l, i: (b, 0, 0)),
            pl.BlockSpec((1, BI, N), lambda b, l, i: (b, i, 0)),
            pl.BlockSpec((1, 1, CMT, N), lambda b, l, i: (b, 2 - l, 0, 0)),
            pl.BlockSpec((1, 1, BI, CMT), lambda b, l, i: (b, 2 - l, i, 0)),
            pl.BlockSpec((1, 1, BI, CMT), lambda b, l, i: (b, 2 - l, i, 0)),
            pl.BlockSpec((1, N, CMT), lambda b, l, i: (b, 0, 0)),
            pl.BlockSpec((1, NB, CMT), lambda b, l, i: (2 - l, 0, 0)),
            pl.BlockSpec((1, NB, NQ, CC), lambda b, l, i: (2 - l, 0, 0, 0)),
            pl.BlockSpec((1, 64, 64), lambda b, l, i: (2 - l, 0, 0)),
            pl.BlockSpec((1, 192, 192), lambda b, l, i: (2 - l, 0, 0)),
            pl.BlockSpec((1, 320, 320), lambda b, l, i: (2 - l, 0, 0)),
        ],
        out_specs=pl.BlockSpec((1, 3, N), lambda b, l, i: (b, 0, 0)),
        out_shape=jax.ShapeDtypeStruct((B, 3, N), jnp.float32),
        scratch_shapes=[pltpu.VMEM((2, N, CMT), jnp.float32)],
    )(posn, post, nmaskf, fts, U, A, gn3, wexp, vexp, lwt0, lwt1, lwt2)


def _prep_weights(params):
    wes, ves, lws, lwts, lbs = [], [], [[], [], []], [[], [], []], []
    for blk in params["blocks"]:
        wes.append(jnp.concatenate(
            [jnp.tile(blk["W"][l], (1, MS[l])) for l in range(3)], axis=1))
        ves.append(jnp.stack(
            [blk["V"][0]] + [blk["V"][1]] * 3 + [blk["V"][2]] * 5, axis=1))
        lbp = []
        for l in range(3):
            M = MS[l]
            perm = (jnp.arange(CC)[None, :] * M + jnp.arange(M)[:, None]).reshape(-1)
            w = blk["lw"][l][perm][:, perm]
            lws[l].append(w)
            lwts[l].append(w.T)
            lbp.append(blk["lb"][l][perm])
        lbs.append(jnp.concatenate(lbp).reshape(1, CMT))
    return dict(
        wexp=jnp.stack(wes), vexp=jnp.stack(ves),
        lw=[jnp.stack(x) for x in lws], lwt=[jnp.stack(x) for x in lwts],
        lb=jnp.stack(lbs))


def kernel(z, pos, neighbor_mask, params):
    B, N, _ = pos.shape
    posn = pos.astype(jnp.float32)
    post = jnp.transpose(posn, (0, 2, 1))
    nmaskf = neighbor_mask.astype(jnp.float32)
    w = _prep_weights(params)
    embp = jnp.concatenate(
        [params["emb"],
         jnp.zeros((128 - params["emb"].shape[0], CC), jnp.float32)], axis=0)
    ow = params["out_w"].reshape(1, CC)
    z3 = z.reshape(B, N, 1).astype(jnp.int32)

    Fins, U, A, e = _fwd_all(z3, embp, posn, post, nmaskf, w["wexp"],
                             w["vexp"], w["lw"][0], w["lw"][1], w["lw"][2],
                             w["lb"], ow)
    E = e[:, 0, 0] + N * params["out_b"][0]

    GN3 = jnp.concatenate(
        [jnp.broadcast_to(params["out_w"][:, 0][None, None, :], (B, N, CC)),
         jnp.zeros((B, N, CMT - CC), jnp.float32)], axis=2)
    FTs = jnp.transpose(Fins, (0, 1, 3, 2))
    gpt = _bwd_all(posn, post, nmaskf, FTs, U, A, GN3, w["wexp"], w["vexp"],
                   w["lwt"][0], w["lwt"][1], w["lwt"][2])
    Fforce = -jnp.transpose(gpt, (0, 2, 1))
    return (E, Fforce)
